# Initial kernel scaffold; baseline (speedup 1.0000x reference)
import jax
import jax.numpy as jnp
from jax.experimental import pallas as pl


def kernel(x, Wk, Wq, Wv, gamma):
    raise NotImplementedError("write your pallas kernel here")


# speedup vs baseline: 4.8150x; 4.8150x over previous
import jax
import jax.numpy as jnp
from jax.experimental import pallas as pl

_T = 128
_D = 64
_TOPK = 8
_ALPHA = 0.1
_BLOCK = 128
_MAXR = 255.0


def _head_body(x_ref, wq_ref, wk_ref, wv_ref, g_ref, out_ref):
    x = x_ref[0]
    q = jnp.dot(x, wq_ref[...], preferred_element_type=jnp.float32)
    k = jnp.dot(x, wk_ref[...], preferred_element_type=jnp.float32)
    v = jnp.dot(x, wv_ref[...], preferred_element_type=jnp.float32)

    s = jnp.dot(q, k.T, preferred_element_type=jnp.float32) * 0.125

    row = jax.lax.broadcasted_iota(jnp.int32, (_T, _T), 0)
    col = jax.lax.broadcasted_iota(jnp.int32, (_T, _T), 1)
    decay = 1.0 - _ALPHA * jnp.abs(row - col).astype(jnp.float32) / _BLOCK
    f = jnp.where(col <= row, jnp.maximum(s, 0.0) * decay, 0.0)

    mean = jnp.mean(f, axis=-1, keepdims=True)
    var = jnp.sum((f - mean) * (f - mean), axis=-1, keepdims=True) / (_T - 1)
    sigma = jnp.sqrt(var)
    m = jnp.max(f, axis=-1, keepdims=True)
    denom = jnp.maximum(m, sigma) + 1e-6

    work = f
    thresh = None
    for _ in range(_TOPK):
        thresh = jnp.max(work, axis=-1, keepdims=True)
        work = jnp.where(work >= thresh, -1.0, work)

    norm = jnp.clip(jnp.floor(_MAXR * f / denom), 0.0, _MAXR)
    wrapped = jnp.where(norm > 127.5, norm - 256.0, norm)
    w = jnp.where(f >= thresh, wrapped, 0.0) * (1.0 / g_ref[0, 0])

    out_ref[0] = jnp.dot(w, v, preferred_element_type=jnp.float32)


def kernel(x, Wk, Wq, Wv, gamma):
    b, t, d = x.shape
    g = jnp.reshape(gamma, (1, 1)).astype(jnp.float32)
    return pl.pallas_call(
        _head_body,
        grid=(b,),
        in_specs=[
            pl.BlockSpec((1, t, d), lambda i: (i, 0, 0)),
            pl.BlockSpec((d, d), lambda i: (0, 0)),
            pl.BlockSpec((d, d), lambda i: (0, 0)),
            pl.BlockSpec((d, d), lambda i: (0, 0)),
            pl.BlockSpec((1, 1), lambda i: (0, 0)),
        ],
        out_specs=pl.BlockSpec((1, t, d), lambda i: (i, 0, 0)),
        out_shape=jax.ShapeDtypeStruct((b, t, d), jnp.float32),
    )(x, Wq, Wk, Wv, g)


# speedup vs baseline: 16.1403x; 3.3521x over previous
import jax
import jax.numpy as jnp
from jax.experimental import pallas as pl

_T = 128
_D = 64
_TOPK = 8
_ALPHA = 0.1
_BLOCK = 128
_MAXR = 255.0


_BB = 8


def _head_body(x_ref, wq_ref, wk_ref, wv_ref, g_ref, out_ref):
    x = x_ref[...].reshape(_BB * _T, _D)
    q = jnp.dot(x, wq_ref[...], preferred_element_type=jnp.float32)
    k = jnp.dot(x, wk_ref[...], preferred_element_type=jnp.float32)
    v = jnp.dot(x, wv_ref[...], preferred_element_type=jnp.float32)
    q = q.reshape(_BB, _T, _D)
    k = k.reshape(_BB, _T, _D)
    v = v.reshape(_BB, _T, _D)

    s = jax.lax.dot_general(
        q, k, (((2,), (2,)), ((0,), (0,))),
        preferred_element_type=jnp.float32) * 0.125

    row = jax.lax.broadcasted_iota(jnp.int32, (1, _T, _T), 1)
    col = jax.lax.broadcasted_iota(jnp.int32, (1, _T, _T), 2)
    decay = 1.0 - _ALPHA * jnp.abs(row - col).astype(jnp.float32) / _BLOCK
    f = jnp.where(col <= row, jnp.maximum(s, 0.0) * decay, 0.0)

    mean = jnp.mean(f, axis=-1, keepdims=True)
    var = jnp.sum((f - mean) * (f - mean), axis=-1, keepdims=True) / (_T - 1)
    sigma = jnp.sqrt(var)
    m = jnp.max(f, axis=-1, keepdims=True)
    denom = jnp.maximum(m, sigma) + 1e-6

    work = f
    thresh = None
    for _ in range(_TOPK):
        thresh = jnp.max(work, axis=-1, keepdims=True)
        work = jnp.where(work >= thresh, -1.0, work)

    norm = jnp.clip(jnp.floor(_MAXR * f / denom), 0.0, _MAXR)
    wrapped = jnp.where(norm > 127.5, norm - 256.0, norm)
    w = jnp.where(f >= thresh, wrapped, 0.0) * (1.0 / g_ref[0, 0])

    out_ref[...] = jax.lax.dot_general(
        w, v, (((2,), (1,)), ((0,), (0,))),
        preferred_element_type=jnp.float32)


def kernel(x, Wk, Wq, Wv, gamma):
    b, t, d = x.shape
    g = jnp.reshape(gamma, (1, 1)).astype(jnp.float32)
    return pl.pallas_call(
        _head_body,
        grid=(b // _BB,),
        in_specs=[
            pl.BlockSpec((_BB, t, d), lambda i: (i, 0, 0)),
            pl.BlockSpec((d, d), lambda i: (0, 0)),
            pl.BlockSpec((d, d), lambda i: (0, 0)),
            pl.BlockSpec((d, d), lambda i: (0, 0)),
            pl.BlockSpec((1, 1), lambda i: (0, 0)),
        ],
        out_specs=pl.BlockSpec((_BB, t, d), lambda i: (i, 0, 0)),
        out_shape=jax.ShapeDtypeStruct((b, t, d), jnp.float32),
    )(x, Wq, Wk, Wv, g)


# speedup vs baseline: 17.2220x; 1.0670x over previous
import jax
import jax.numpy as jnp
from jax.experimental import pallas as pl

_T = 128
_D = 64
_TOPK = 8
_ALPHA = 0.1
_BLOCK = 128
_MAXR = 255.0


_BB = 16


def _head_body(x_ref, wq_ref, wk_ref, wv_ref, g_ref, out_ref):
    x = x_ref[...].reshape(_BB * _T, _D)
    q = jnp.dot(x, wq_ref[...], preferred_element_type=jnp.float32)
    k = jnp.dot(x, wk_ref[...], preferred_element_type=jnp.float32)
    v = jnp.dot(x, wv_ref[...], preferred_element_type=jnp.float32)
    q = q.reshape(_BB, _T, _D)
    k = k.reshape(_BB, _T, _D)
    v = v.reshape(_BB, _T, _D)

    s = jax.lax.dot_general(
        q, k, (((2,), (2,)), ((0,), (0,))),
        preferred_element_type=jnp.float32) * 0.125

    row = jax.lax.broadcasted_iota(jnp.int32, (1, _T, _T), 1)
    col = jax.lax.broadcasted_iota(jnp.int32, (1, _T, _T), 2)
    decay = 1.0 - _ALPHA * jnp.abs(row - col).astype(jnp.float32) / _BLOCK
    f = jnp.where(col <= row, jnp.maximum(s, 0.0) * decay, 0.0)

    mean = jnp.mean(f, axis=-1, keepdims=True)
    var = jnp.sum((f - mean) * (f - mean), axis=-1, keepdims=True) / (_T - 1)
    sigma = jnp.sqrt(var)
    m = jnp.max(f, axis=-1, keepdims=True)
    denom = jnp.maximum(m, sigma) + 1e-6

    work = f
    thresh = None
    for _ in range(_TOPK):
        thresh = jnp.max(work, axis=-1, keepdims=True)
        work = jnp.where(work >= thresh, -1.0, work)

    norm = jnp.clip(jnp.floor(_MAXR * f / denom), 0.0, _MAXR)
    wrapped = jnp.where(norm > 127.5, norm - 256.0, norm)
    w = jnp.where(f >= thresh, wrapped, 0.0) * (1.0 / g_ref[0, 0])

    out_ref[...] = jax.lax.dot_general(
        w, v, (((2,), (1,)), ((0,), (0,))),
        preferred_element_type=jnp.float32)


def kernel(x, Wk, Wq, Wv, gamma):
    b, t, d = x.shape
    g = jnp.reshape(gamma, (1, 1)).astype(jnp.float32)
    return pl.pallas_call(
        _head_body,
        grid=(b // _BB,),
        in_specs=[
            pl.BlockSpec((_BB, t, d), lambda i: (i, 0, 0)),
            pl.BlockSpec((d, d), lambda i: (0, 0)),
            pl.BlockSpec((d, d), lambda i: (0, 0)),
            pl.BlockSpec((d, d), lambda i: (0, 0)),
            pl.BlockSpec((1, 1), lambda i: (0, 0)),
        ],
        out_specs=pl.BlockSpec((_BB, t, d), lambda i: (i, 0, 0)),
        out_shape=jax.ShapeDtypeStruct((b, t, d), jnp.float32),
    )(x, Wq, Wk, Wv, g)


# speedup vs baseline: 17.7977x; 1.0334x over previous
import jax
import jax.numpy as jnp
from jax.experimental import pallas as pl

_T = 128
_D = 64
_TOPK = 8
_ALPHA = 0.1
_BLOCK = 128
_MAXR = 255.0


_BB = 32


def _head_body(x_ref, wq_ref, wk_ref, wv_ref, g_ref, out_ref):
    x = x_ref[...].reshape(_BB * _T, _D)
    q = jnp.dot(x, wq_ref[...], preferred_element_type=jnp.float32)
    k = jnp.dot(x, wk_ref[...], preferred_element_type=jnp.float32)
    v = jnp.dot(x, wv_ref[...], preferred_element_type=jnp.float32)
    q = q.reshape(_BB, _T, _D)
    k = k.reshape(_BB, _T, _D)
    v = v.reshape(_BB, _T, _D)

    s = jax.lax.dot_general(
        q, k, (((2,), (2,)), ((0,), (0,))),
        preferred_element_type=jnp.float32) * 0.125

    row = jax.lax.broadcasted_iota(jnp.int32, (1, _T, _T), 1)
    col = jax.lax.broadcasted_iota(jnp.int32, (1, _T, _T), 2)
    decay = 1.0 - _ALPHA * jnp.abs(row - col).astype(jnp.float32) / _BLOCK
    f = jnp.where(col <= row, jnp.maximum(s, 0.0) * decay, 0.0)

    mean = jnp.mean(f, axis=-1, keepdims=True)
    var = jnp.sum((f - mean) * (f - mean), axis=-1, keepdims=True) / (_T - 1)
    sigma = jnp.sqrt(var)
    m = jnp.max(f, axis=-1, keepdims=True)
    denom = jnp.maximum(m, sigma) + 1e-6

    work = f
    thresh = None
    for _ in range(_TOPK):
        thresh = jnp.max(work, axis=-1, keepdims=True)
        work = jnp.where(work >= thresh, -1.0, work)

    norm = jnp.clip(jnp.floor(_MAXR * f / denom), 0.0, _MAXR)
    wrapped = jnp.where(norm > 127.5, norm - 256.0, norm)
    w = jnp.where(f >= thresh, wrapped, 0.0) * (1.0 / g_ref[0, 0])

    out_ref[...] = jax.lax.dot_general(
        w, v, (((2,), (1,)), ((0,), (0,))),
        preferred_element_type=jnp.float32)


def kernel(x, Wk, Wq, Wv, gamma):
    b, t, d = x.shape
    g = jnp.reshape(gamma, (1, 1)).astype(jnp.float32)
    return pl.pallas_call(
        _head_body,
        grid=(b // _BB,),
        in_specs=[
            pl.BlockSpec((_BB, t, d), lambda i: (i, 0, 0)),
            pl.BlockSpec((d, d), lambda i: (0, 0)),
            pl.BlockSpec((d, d), lambda i: (0, 0)),
            pl.BlockSpec((d, d), lambda i: (0, 0)),
            pl.BlockSpec((1, 1), lambda i: (0, 0)),
        ],
        out_specs=pl.BlockSpec((_BB, t, d), lambda i: (i, 0, 0)),
        out_shape=jax.ShapeDtypeStruct((b, t, d), jnp.float32),
    )(x, Wq, Wk, Wv, g)


# speedup vs baseline: 18.0495x; 1.0141x over previous
import jax
import jax.numpy as jnp
from jax.experimental import pallas as pl

_T = 128
_D = 64
_TOPK = 8
_ALPHA = 0.1
_BLOCK = 128
_MAXR = 255.0


_BB = 64


def _head_body(x_ref, wq_ref, wk_ref, wv_ref, g_ref, out_ref):
    x = x_ref[...].reshape(_BB * _T, _D)
    q = jnp.dot(x, wq_ref[...], preferred_element_type=jnp.float32)
    k = jnp.dot(x, wk_ref[...], preferred_element_type=jnp.float32)
    v = jnp.dot(x, wv_ref[...], preferred_element_type=jnp.float32)
    q = q.reshape(_BB, _T, _D)
    k = k.reshape(_BB, _T, _D)
    v = v.reshape(_BB, _T, _D)

    s = jax.lax.dot_general(
        q, k, (((2,), (2,)), ((0,), (0,))),
        preferred_element_type=jnp.float32) * 0.125

    row = jax.lax.broadcasted_iota(jnp.int32, (1, _T, _T), 1)
    col = jax.lax.broadcasted_iota(jnp.int32, (1, _T, _T), 2)
    decay = 1.0 - _ALPHA * jnp.abs(row - col).astype(jnp.float32) / _BLOCK
    f = jnp.where(col <= row, jnp.maximum(s, 0.0) * decay, 0.0)

    mean = jnp.mean(f, axis=-1, keepdims=True)
    var = jnp.sum((f - mean) * (f - mean), axis=-1, keepdims=True) / (_T - 1)
    sigma = jnp.sqrt(var)
    m = jnp.max(f, axis=-1, keepdims=True)
    denom = jnp.maximum(m, sigma) + 1e-6

    work = f
    thresh = None
    for _ in range(_TOPK):
        thresh = jnp.max(work, axis=-1, keepdims=True)
        work = jnp.where(work >= thresh, -1.0, work)

    norm = jnp.clip(jnp.floor(_MAXR * f / denom), 0.0, _MAXR)
    wrapped = jnp.where(norm > 127.5, norm - 256.0, norm)
    w = jnp.where(f >= thresh, wrapped, 0.0) * (1.0 / g_ref[0, 0])

    out_ref[...] = jax.lax.dot_general(
        w, v, (((2,), (1,)), ((0,), (0,))),
        preferred_element_type=jnp.float32)


def kernel(x, Wk, Wq, Wv, gamma):
    b, t, d = x.shape
    g = jnp.reshape(gamma, (1, 1)).astype(jnp.float32)
    return pl.pallas_call(
        _head_body,
        grid=(b // _BB,),
        in_specs=[
            pl.BlockSpec((_BB, t, d), lambda i: (i, 0, 0)),
            pl.BlockSpec((d, d), lambda i: (0, 0)),
            pl.BlockSpec((d, d), lambda i: (0, 0)),
            pl.BlockSpec((d, d), lambda i: (0, 0)),
            pl.BlockSpec((1, 1), lambda i: (0, 0)),
        ],
        out_specs=pl.BlockSpec((_BB, t, d), lambda i: (i, 0, 0)),
        out_shape=jax.ShapeDtypeStruct((b, t, d), jnp.float32),
    )(x, Wq, Wk, Wv, g)


# speedup vs baseline: 18.0800x; 1.0017x over previous
import jax
import jax.numpy as jnp
from jax.experimental import pallas as pl

_T = 128
_D = 64
_TOPK = 8
_ALPHA = 0.1
_BLOCK = 128
_MAXR = 255.0

_BB = 64


def _head_body(x_ref, wq_ref, wk_ref, wv_ref, d8m_ref, g_ref, out_ref):
    x = x_ref[...].reshape(_BB * _T, _D)
    q = jnp.dot(x, wq_ref[...], preferred_element_type=jnp.float32)
    k = jnp.dot(x, wk_ref[...], preferred_element_type=jnp.float32)
    v = jnp.dot(x, wv_ref[...], preferred_element_type=jnp.float32)
    q = q.reshape(_BB, _T, _D)
    k = k.reshape(_BB, _T, _D)
    v = v.reshape(_BB, _T, _D) * (1.0 / g_ref[0, 0])

    s = jax.lax.dot_general(
        q, k, (((2,), (2,)), ((0,), (0,))),
        preferred_element_type=jnp.float32)

    f = jnp.maximum(s, 0.0) * d8m_ref[...][None]

    mean = jnp.mean(f, axis=-1, keepdims=True)
    sumsq = jnp.sum(f * f, axis=-1, keepdims=True)
    var = jnp.maximum(sumsq - mean * mean * _T, 0.0) / (_T - 1)
    sigma = jnp.sqrt(var)

    thresh = jnp.max(f, axis=-1, keepdims=True)
    m = thresh
    for _ in range(_TOPK - 1):
        thresh = jnp.max(jnp.where(f >= thresh, -1.0, f),
                         axis=-1, keepdims=True)

    denom = jnp.maximum(m, sigma) + 1e-6
    r = _MAXR / denom
    norm = jnp.floor(f * r)
    w = jnp.where(f >= thresh, norm - jnp.where(norm > 127.5, 256.0, 0.0),
                  0.0)

    out_ref[...] = jax.lax.dot_general(
        w, v, (((2,), (1,)), ((0,), (0,))),
        preferred_element_type=jnp.float32)


def kernel(x, Wk, Wq, Wv, gamma):
    b, t, d = x.shape
    g = jnp.reshape(gamma, (1, 1)).astype(jnp.float32)
    row = jax.lax.broadcasted_iota(jnp.int32, (t, t), 0)
    col = jax.lax.broadcasted_iota(jnp.int32, (t, t), 1)
    decay = 1.0 - _ALPHA * jnp.abs(row - col).astype(jnp.float32) / _BLOCK
    d8m = jnp.where(col <= row, decay * 0.125, 0.0)
    return pl.pallas_call(
        _head_body,
        grid=(b // _BB,),
        in_specs=[
            pl.BlockSpec((_BB, t, d), lambda i: (i, 0, 0)),
            pl.BlockSpec((d, d), lambda i: (0, 0)),
            pl.BlockSpec((d, d), lambda i: (0, 0)),
            pl.BlockSpec((d, d), lambda i: (0, 0)),
            pl.BlockSpec((t, t), lambda i: (0, 0)),
            pl.BlockSpec((1, 1), lambda i: (0, 0)),
        ],
        out_specs=pl.BlockSpec((_BB, t, d), lambda i: (i, 0, 0)),
        out_shape=jax.ShapeDtypeStruct((b, t, d), jnp.float32),
    )(x, Wq, Wk, Wv, d8m, g)
